# Initial kernel scaffold; baseline (speedup 1.0000x reference)
import jax
import jax.numpy as jnp
from jax import lax
from jax.experimental import pallas as pl
from jax.experimental.pallas import tpu as pltpu
from jax.experimental.pallas import tpu_sc as plsc

N = 10000
E = 320000
HID = 64
NUM_GRAPHS = 128
NC = 2
NS = 16
EW = E // (NC * NS)
CHUNK = 1000
ROWS_PER_TILE = N // NS
BLK = 1000
GRID = N // BLK


def _mm_body(h_ref, w_ref, b_ref, o_ref):
    o_ref[...] = (
        jnp.dot(h_ref[...], w_ref[...], preferred_element_type=jnp.float32)
        + b_ref[...]
    )


def _tc_matmul(h, w, b):
    m, k = h.shape
    n = w.shape[1]
    return pl.pallas_call(
        _mm_body,
        grid=(m // BLK,),
        in_specs=[
            pl.BlockSpec((BLK, k), lambda i: (i, 0)),
            pl.BlockSpec((k, n), lambda i: (0, 0)),
            pl.BlockSpec((1, n), lambda i: (0, 0)),
        ],
        out_specs=pl.BlockSpec((BLK, n), lambda i: (i, 0)),
        out_shape=jax.ShapeDtypeStruct((m, n), jnp.float32),
    )(h, w, b)


def _combine_mm_body(aa_ref, ab_ref, da_ref, db_ref, r_ref, w_ref, b_ref, o_ref):
    deg = (da_ref[...] + db_ref[...])[:, 0:1]
    mean = (aa_ref[...] + ab_ref[...]) / jnp.maximum(deg, 1.0)
    h = jax.nn.relu(mean + r_ref[...])
    o_ref[...] = (
        jnp.dot(h, w_ref[...], preferred_element_type=jnp.float32) + b_ref[...]
    )


def _tc_combine_matmul(agg_a, agg_b, deg_a, deg_b, r, w, b):
    n = w.shape[1]
    return pl.pallas_call(
        _combine_mm_body,
        grid=(GRID,),
        in_specs=[
            pl.BlockSpec((BLK, HID), lambda i: (i, 0)),
            pl.BlockSpec((BLK, HID), lambda i: (i, 0)),
            pl.BlockSpec((BLK, 16), lambda i: (i, 0)),
            pl.BlockSpec((BLK, 16), lambda i: (i, 0)),
            pl.BlockSpec((BLK, HID), lambda i: (i, 0)),
            pl.BlockSpec((HID, n), lambda i: (0, 0)),
            pl.BlockSpec((1, n), lambda i: (0, 0)),
        ],
        out_specs=pl.BlockSpec((BLK, n), lambda i: (i, 0)),
        out_shape=jax.ShapeDtypeStruct((N, n), jnp.float32),
    )(agg_a, agg_b, deg_a, deg_b, r, w, b)


def _pool_body(aa_ref, ab_ref, da_ref, db_ref, r_ref, bat_ref, wl_ref, bl_ref,
               o_ref, gacc, cacc):
    i = pl.program_id(0)

    @pl.when(i == 0)
    def _():
        gacc[...] = jnp.zeros_like(gacc)
        cacc[...] = jnp.zeros_like(cacc)

    deg = (da_ref[...] + db_ref[...])[:, 0:1]
    mean = (aa_ref[...] + ab_ref[...]) / jnp.maximum(deg, 1.0)
    h = jax.nn.relu(mean + r_ref[...])
    bat = bat_ref[...].reshape(1, BLK)
    gid = lax.broadcasted_iota(jnp.float32, (NUM_GRAPHS, BLK), 0)
    onehot_t = (gid == bat).astype(jnp.float32)
    gacc[...] += jnp.dot(onehot_t, h, preferred_element_type=jnp.float32)
    cacc[...] += jnp.sum(onehot_t, axis=1, keepdims=True)

    @pl.when(i == GRID - 1)
    def _():
        g = gacc[...] / jnp.maximum(cacc[...], 1.0)
        o_ref[...] = (
            jnp.dot(g, wl_ref[...], preferred_element_type=jnp.float32)
            + bl_ref[...]
        )


def _tc_pool(agg_a, agg_b, deg_a, deg_b, r, batf, w_lin, b_lin):
    return pl.pallas_call(
        _pool_body,
        grid=(GRID,),
        in_specs=[
            pl.BlockSpec((BLK, HID), lambda i: (i, 0)),
            pl.BlockSpec((BLK, HID), lambda i: (i, 0)),
            pl.BlockSpec((BLK, 16), lambda i: (i, 0)),
            pl.BlockSpec((BLK, 16), lambda i: (i, 0)),
            pl.BlockSpec((BLK, HID), lambda i: (i, 0)),
            pl.BlockSpec((1, 1, BLK), lambda i: (i, 0, 0)),
            pl.BlockSpec((HID, 1), lambda i: (0, 0)),
            pl.BlockSpec((1, 1), lambda i: (0, 0)),
        ],
        out_specs=pl.BlockSpec((NUM_GRAPHS, 1), lambda i: (0, 0)),
        out_shape=jax.ShapeDtypeStruct((NUM_GRAPHS, 1), jnp.float32),
        scratch_shapes=[
            pltpu.VMEM((NUM_GRAPHS, HID), jnp.float32),
            pltpu.VMEM((NUM_GRAPHS, 1), jnp.float32),
        ],
    )(agg_a, agg_b, deg_a, deg_b, r, batf, w_lin, b_lin)


def _sc_agg_call(p, src, dst, zeros64, zeros16, ones, with_deg):
    mesh = plsc.VectorSubcoreMesh(core_axis_name="c", subcore_axis_name="s")

    out_type = [jax.ShapeDtypeStruct((NC * N, HID), jnp.float32)]
    scratch = [
        pltpu.VMEM((CHUNK,), jnp.int32),
        pltpu.VMEM((CHUNK,), jnp.int32),
        pltpu.VMEM((CHUNK, HID), jnp.float32),
        pltpu.VMEM_SHARED((N, HID), jnp.float32),
        pltpu.SemaphoreType.DMA,
    ]
    if with_deg:
        out_type.append(jax.ShapeDtypeStruct((NC * N, 16), jnp.float32))
        scratch += [
            pltpu.VMEM((CHUNK, 16), jnp.float32),
            pltpu.VMEM_SHARED((N, 16), jnp.float32),
        ]

    def body(*refs):
        if with_deg:
            (p_h, src_h, dst_h, z64_h, z16_h, ones_h,
             out_h, deg_h, srcv, dstv, msgv, acc, sem, onesv, dacc) = refs
        else:
            (p_h, src_h, dst_h, z64_h,
             out_h, srcv, dstv, msgv, acc, sem) = refs
        c = lax.axis_index("c")
        s = lax.axis_index("s")
        wid = c * NS + s
        base = wid * EW
        r0 = s * ROWS_PER_TILE

        pltpu.sync_copy(z64_h.at[pl.ds(r0, ROWS_PER_TILE)],
                        acc.at[pl.ds(r0, ROWS_PER_TILE)])
        if with_deg:
            pltpu.sync_copy(z16_h.at[pl.ds(r0, ROWS_PER_TILE)],
                            dacc.at[pl.ds(r0, ROWS_PER_TILE)])
            pltpu.sync_copy(ones_h, onesv)
        plsc.subcore_barrier()

        def chunk(i, carry):
            off = base + i * CHUNK
            pltpu.sync_copy(src_h.at[pl.ds(off, CHUNK)], srcv)
            pltpu.sync_copy(dst_h.at[pl.ds(off, CHUNK)], dstv)
            pltpu.async_copy(p_h.at[srcv], msgv, sem).wait()
            pltpu.sync_copy(msgv, acc.at[dstv], add=True)
            if with_deg:
                pltpu.sync_copy(onesv, dacc.at[dstv], add=True)
            return carry

        lax.fori_loop(0, EW // CHUNK, chunk, 0)
        plsc.subcore_barrier()

        orow = c * N + r0
        pltpu.sync_copy(acc.at[pl.ds(r0, ROWS_PER_TILE)],
                        out_h.at[pl.ds(orow, ROWS_PER_TILE)])
        if with_deg:
            pltpu.sync_copy(dacc.at[pl.ds(r0, ROWS_PER_TILE)],
                            deg_h.at[pl.ds(orow, ROWS_PER_TILE)])

    fn = pl.kernel(body, out_type=out_type, mesh=mesh, scratch_types=scratch)
    if with_deg:
        return fn(p, src, dst, zeros64, zeros16, ones)
    return fn(p, src, dst, zeros64)


def kernel(x, pos, edge_index, batch, W1_l, W1_r, b1, W2_l, W2_r, b2,
           W_lin, b_lin):
    f32 = jnp.float32
    src = edge_index[0].astype(jnp.int32)
    dst = edge_index[1].astype(jnp.int32)

    h0 = jnp.concatenate([x, pos], axis=1)
    d_in = h0.shape[1]
    kpad = 136
    h0p = jnp.pad(h0, ((0, 0), (0, kpad - d_in)))
    w1 = jnp.pad(jnp.concatenate([W1_l, W1_r], axis=1),
                 ((0, kpad - d_in), (0, 0)))
    b1cat = jnp.concatenate([jnp.zeros((HID,), f32), b1]).reshape(1, 2 * HID)
    out1 = _tc_matmul(h0p, w1, b1cat)
    p1 = out1[:, :HID]
    r1 = out1[:, HID:]

    zeros64 = jnp.zeros((N, HID), f32)
    zeros16 = jnp.zeros((N, 16), f32)
    ones = jnp.ones((CHUNK, 16), f32)

    agg1, deg = _sc_agg_call(p1, src, dst, zeros64, zeros16, ones, True)
    deg_a, deg_b = deg[:N], deg[N:]

    w2 = jnp.concatenate([W2_l, W2_r], axis=1)
    b2cat = jnp.concatenate([jnp.zeros((HID,), f32), b2]).reshape(1, 2 * HID)
    out2 = _tc_combine_matmul(agg1[:N], agg1[N:], deg_a, deg_b, r1, w2, b2cat)
    p2 = out2[:, :HID]
    r2 = out2[:, HID:]

    (agg2,) = _sc_agg_call(p2, src, dst, zeros64, None, None, False)

    batf = batch.astype(f32).reshape(GRID, 1, BLK)
    out = _tc_pool(agg2[:N], agg2[N:], deg_a, deg_b, r2, batf,
                   W_lin, b_lin.reshape(1, 1))
    return out


# speedup vs baseline: 13.0334x; 13.0334x over previous
import jax
import jax.numpy as jnp
from jax import lax
from jax.experimental import pallas as pl
from jax.experimental.pallas import tpu as pltpu
from jax.experimental.pallas import tpu_sc as plsc

N = 10000
E = 320000
HID = 64
NUM_GRAPHS = 128
NC = 2
NS = 16
EW = E // (NC * NS)
CHUNK = 1000
NP = 10240
ROWS_PER_TILE = NP // NS
BLK = 1000
GRID = N // BLK


def _mm_body(h_ref, w_ref, b_ref, o_ref):
    o_ref[...] = (
        jnp.dot(h_ref[...], w_ref[...], preferred_element_type=jnp.float32)
        + b_ref[...]
    )


def _tc_matmul(h, w, b):
    m, k = h.shape
    n = w.shape[1]
    return pl.pallas_call(
        _mm_body,
        grid=(m // BLK,),
        in_specs=[
            pl.BlockSpec((BLK, k), lambda i: (i, 0)),
            pl.BlockSpec((k, n), lambda i: (0, 0)),
            pl.BlockSpec((1, n), lambda i: (0, 0)),
        ],
        out_specs=pl.BlockSpec((BLK, n), lambda i: (i, 0)),
        out_shape=jax.ShapeDtypeStruct((m, n), jnp.float32),
    )(h, w, b)


def _combine_mm_body(aa_ref, ab_ref, da_ref, db_ref, r_ref, w_ref, b_ref, o_ref):
    deg = (da_ref[...] + db_ref[...])[:, 0:1]
    mean = (aa_ref[...] + ab_ref[...]) / jnp.maximum(deg, 1.0)
    h = jax.nn.relu(mean + r_ref[...])
    o_ref[...] = (
        jnp.dot(h, w_ref[...], preferred_element_type=jnp.float32) + b_ref[...]
    )


def _tc_combine_matmul(agg_a, agg_b, deg_a, deg_b, r, w, b):
    n = w.shape[1]
    return pl.pallas_call(
        _combine_mm_body,
        grid=(GRID,),
        in_specs=[
            pl.BlockSpec((BLK, HID), lambda i: (i, 0)),
            pl.BlockSpec((BLK, HID), lambda i: (i, 0)),
            pl.BlockSpec((BLK, 16), lambda i: (i, 0)),
            pl.BlockSpec((BLK, 16), lambda i: (i, 0)),
            pl.BlockSpec((BLK, HID), lambda i: (i, 0)),
            pl.BlockSpec((HID, n), lambda i: (0, 0)),
            pl.BlockSpec((1, n), lambda i: (0, 0)),
        ],
        out_specs=pl.BlockSpec((BLK, n), lambda i: (i, 0)),
        out_shape=jax.ShapeDtypeStruct((N, n), jnp.float32),
    )(agg_a, agg_b, deg_a, deg_b, r, w, b)


def _pool_body(aa_ref, ab_ref, da_ref, db_ref, r_ref, bat_ref, wl_ref, bl_ref,
               o_ref, gacc, cacc):
    i = pl.program_id(0)

    @pl.when(i == 0)
    def _():
        gacc[...] = jnp.zeros_like(gacc)
        cacc[...] = jnp.zeros_like(cacc)

    deg = (da_ref[...] + db_ref[...])[:, 0:1]
    mean = (aa_ref[...] + ab_ref[...]) / jnp.maximum(deg, 1.0)
    h = jax.nn.relu(mean + r_ref[...])
    bat = bat_ref[...].reshape(1, BLK)
    gid = lax.broadcasted_iota(jnp.int32, (NUM_GRAPHS, BLK), 0)
    onehot_t = (gid == bat).astype(jnp.float32)
    gacc[...] += jnp.dot(onehot_t, h, preferred_element_type=jnp.float32)
    cacc[...] += jnp.sum(onehot_t, axis=1, keepdims=True)

    @pl.when(i == GRID - 1)
    def _():
        g = gacc[...] / jnp.maximum(cacc[...], 1.0)
        o_ref[...] = (
            jnp.dot(g, wl_ref[...], preferred_element_type=jnp.float32)
            + bl_ref[...]
        )


def _tc_pool(agg_a, agg_b, deg_a, deg_b, r, batf, w_lin, b_lin):
    return pl.pallas_call(
        _pool_body,
        grid=(GRID,),
        in_specs=[
            pl.BlockSpec((BLK, HID), lambda i: (i, 0)),
            pl.BlockSpec((BLK, HID), lambda i: (i, 0)),
            pl.BlockSpec((BLK, 16), lambda i: (i, 0)),
            pl.BlockSpec((BLK, 16), lambda i: (i, 0)),
            pl.BlockSpec((BLK, HID), lambda i: (i, 0)),
            pl.BlockSpec((1, 1, BLK), lambda i: (i, 0, 0)),
            pl.BlockSpec((HID, 1), lambda i: (0, 0)),
            pl.BlockSpec((1, 1), lambda i: (0, 0)),
        ],
        out_specs=pl.BlockSpec((NUM_GRAPHS, 1), lambda i: (0, 0)),
        out_shape=jax.ShapeDtypeStruct((NUM_GRAPHS, 1), jnp.float32),
        scratch_shapes=[
            pltpu.VMEM((NUM_GRAPHS, HID), jnp.float32),
            pltpu.VMEM((NUM_GRAPHS, 1), jnp.float32),
        ],
    )(agg_a, agg_b, deg_a, deg_b, r, batf, w_lin, b_lin)


def _sc_agg_call(p, src, dst, zeros64):
    mesh = plsc.VectorSubcoreMesh(core_axis_name="c", subcore_axis_name="s",
                                  num_cores=NC, num_subcores=NS)

    def body(p_h, src_h, dst_h, z64_h, out_h, srcv, dstv, msgv, acc, sem):
        c = lax.axis_index("c")
        s = lax.axis_index("s")
        wid = c * NS + s
        base = wid * EW
        r0 = s * ROWS_PER_TILE

        pltpu.sync_copy(z64_h.at[pl.ds(r0, ROWS_PER_TILE)],
                        acc.at[pl.ds(r0, ROWS_PER_TILE)])
        plsc.subcore_barrier()

        def chunk(i, carry):
            off = base + i * CHUNK
            pltpu.sync_copy(src_h.at[pl.ds(off, CHUNK)], srcv)
            pltpu.sync_copy(dst_h.at[pl.ds(off, CHUNK)], dstv)
            pltpu.async_copy(p_h.at[srcv], msgv, sem).wait()
            pltpu.sync_copy(msgv, acc.at[dstv], add=True)
            return carry

        lax.fori_loop(0, EW // CHUNK, chunk, 0)
        plsc.subcore_barrier()

        orow = c * NP + r0
        pltpu.sync_copy(acc.at[pl.ds(r0, ROWS_PER_TILE)],
                        out_h.at[pl.ds(orow, ROWS_PER_TILE)])

    fn = pl.kernel(
        body,
        out_type=[jax.ShapeDtypeStruct((NC * NP, HID), jnp.float32)],
        mesh=mesh,
        scratch_types=[
            pltpu.VMEM((CHUNK,), jnp.int32),
            pltpu.VMEM((CHUNK,), jnp.int32),
            pltpu.VMEM((CHUNK, HID), jnp.float32),
            pltpu.VMEM_SHARED((NP, HID), jnp.float32),
            pltpu.SemaphoreType.DMA,
        ],
        compiler_params=pltpu.CompilerParams(use_tc_tiling_on_sc=False),
    )
    (out,) = fn(p, src, dst, zeros64)
    return out


DCHUNK = 2000


def _sc_degree_call(dst, zeros16, ones):
    mesh = plsc.VectorSubcoreMesh(core_axis_name="c", subcore_axis_name="s",
                                  num_cores=NC, num_subcores=NS)

    def body(dst_h, z16_h, ones_h, deg_h, dstv, onesv, dacc):
        c = lax.axis_index("c")
        s = lax.axis_index("s")
        wid = c * NS + s
        base = wid * EW
        r0 = s * ROWS_PER_TILE

        pltpu.sync_copy(z16_h.at[pl.ds(r0, ROWS_PER_TILE)],
                        dacc.at[pl.ds(r0, ROWS_PER_TILE)])
        pltpu.sync_copy(ones_h, onesv)
        plsc.subcore_barrier()

        def chunk(i, carry):
            off = base + i * DCHUNK
            pltpu.sync_copy(dst_h.at[pl.ds(off, DCHUNK)], dstv)
            pltpu.sync_copy(onesv, dacc.at[dstv], add=True)
            return carry

        lax.fori_loop(0, EW // DCHUNK, chunk, 0)
        plsc.subcore_barrier()

        orow = c * NP + r0
        pltpu.sync_copy(dacc.at[pl.ds(r0, ROWS_PER_TILE)],
                        deg_h.at[pl.ds(orow, ROWS_PER_TILE)])

    fn = pl.kernel(
        body,
        out_type=[jax.ShapeDtypeStruct((NC * NP, 16), jnp.float32)],
        mesh=mesh,
        scratch_types=[
            pltpu.VMEM((DCHUNK,), jnp.int32),
            pltpu.VMEM((DCHUNK, 16), jnp.float32),
            pltpu.VMEM_SHARED((NP, 16), jnp.float32),
        ],
        compiler_params=pltpu.CompilerParams(use_tc_tiling_on_sc=False),
    )
    (deg,) = fn(dst, zeros16, ones)
    return deg


def kernel(x, pos, edge_index, batch, W1_l, W1_r, b1, W2_l, W2_r, b2,
           W_lin, b_lin):
    f32 = jnp.float32
    src = edge_index[0].astype(jnp.int32)
    dst = edge_index[1].astype(jnp.int32)

    h0 = jnp.concatenate([x, pos], axis=1)
    d_in = h0.shape[1]
    kpad = 136
    h0p = jnp.pad(h0, ((0, 0), (0, kpad - d_in)))
    w1 = jnp.pad(jnp.concatenate([W1_l, W1_r], axis=1),
                 ((0, kpad - d_in), (0, 0)))
    b1cat = jnp.concatenate([jnp.zeros((HID,), f32), b1]).reshape(1, 2 * HID)
    out1 = _tc_matmul(h0p, w1, b1cat)
    p1 = out1[:, :HID]
    r1 = out1[:, HID:]

    zeros64 = jnp.zeros((NP, HID), f32)
    zeros16 = jnp.zeros((NP, 16), f32)
    ones = jnp.ones((DCHUNK, 16), f32)

    deg = _sc_degree_call(dst, zeros16, ones)
    deg_a, deg_b = deg[:N], deg[NP:NP + N]

    agg1 = _sc_agg_call(p1, src, dst, zeros64)

    w2 = jnp.concatenate([W2_l, W2_r], axis=1)
    b2cat = jnp.concatenate([jnp.zeros((HID,), f32), b2]).reshape(1, 2 * HID)
    out2 = _tc_combine_matmul(agg1[:N], agg1[NP:NP + N], deg_a, deg_b, r1,
                              w2, b2cat)
    p2 = out2[:, :HID]
    r2 = out2[:, HID:]

    agg2 = _sc_agg_call(p2, src, dst, zeros64)

    batf = batch.astype(jnp.int32).reshape(GRID, 1, BLK)
    out = _tc_pool(agg2[:N], agg2[NP:NP + N], deg_a, deg_b, r2, batf,
                   W_lin, b_lin.reshape(1, 1))
    return out


# speedup vs baseline: 14.5550x; 1.1167x over previous
import jax
import jax.numpy as jnp
from jax import lax
from jax.experimental import pallas as pl
from jax.experimental.pallas import tpu as pltpu
from jax.experimental.pallas import tpu_sc as plsc

N = 10000
E = 320000
HID = 64
NUM_GRAPHS = 128
NC = 2
NS = 16
EW = E // (NC * NS)
CHUNK = 400
NCH = EW // CHUNK
NP = 10240
ROWS_PER_TILE = NP // NS
BLK = 1000
GRID = N // BLK


def _mm_body(h_ref, w_ref, b_ref, o_ref):
    o_ref[...] = (
        jnp.dot(h_ref[...], w_ref[...], preferred_element_type=jnp.float32)
        + b_ref[...]
    )


def _tc_matmul(h, w, b):
    m, k = h.shape
    n = w.shape[1]
    return pl.pallas_call(
        _mm_body,
        grid=(m // BLK,),
        in_specs=[
            pl.BlockSpec((BLK, k), lambda i: (i, 0)),
            pl.BlockSpec((k, n), lambda i: (0, 0)),
            pl.BlockSpec((1, n), lambda i: (0, 0)),
        ],
        out_specs=pl.BlockSpec((BLK, n), lambda i: (i, 0)),
        out_shape=jax.ShapeDtypeStruct((m, n), jnp.float32),
    )(h, w, b)


def _combine_mm_body(aa_ref, ab_ref, da_ref, db_ref, r_ref, w_ref, b_ref, o_ref):
    deg = (da_ref[...] + db_ref[...])[:, 0:1]
    mean = (aa_ref[...] + ab_ref[...]) / jnp.maximum(deg, 1.0)
    h = jax.nn.relu(mean + r_ref[...])
    o_ref[...] = (
        jnp.dot(h, w_ref[...], preferred_element_type=jnp.float32) + b_ref[...]
    )


def _tc_combine_matmul(agg_a, agg_b, deg_a, deg_b, r, w, b):
    n = w.shape[1]
    return pl.pallas_call(
        _combine_mm_body,
        grid=(GRID,),
        in_specs=[
            pl.BlockSpec((BLK, HID), lambda i: (i, 0)),
            pl.BlockSpec((BLK, HID), lambda i: (i, 0)),
            pl.BlockSpec((BLK, 16), lambda i: (i, 0)),
            pl.BlockSpec((BLK, 16), lambda i: (i, 0)),
            pl.BlockSpec((BLK, HID), lambda i: (i, 0)),
            pl.BlockSpec((HID, n), lambda i: (0, 0)),
            pl.BlockSpec((1, n), lambda i: (0, 0)),
        ],
        out_specs=pl.BlockSpec((BLK, n), lambda i: (i, 0)),
        out_shape=jax.ShapeDtypeStruct((N, n), jnp.float32),
    )(agg_a, agg_b, deg_a, deg_b, r, w, b)


def _pool_body(aa_ref, ab_ref, da_ref, db_ref, r_ref, bat_ref, wl_ref, bl_ref,
               o_ref, gacc, cacc):
    i = pl.program_id(0)

    @pl.when(i == 0)
    def _():
        gacc[...] = jnp.zeros_like(gacc)
        cacc[...] = jnp.zeros_like(cacc)

    deg = (da_ref[...] + db_ref[...])[:, 0:1]
    mean = (aa_ref[...] + ab_ref[...]) / jnp.maximum(deg, 1.0)
    h = jax.nn.relu(mean + r_ref[...])
    bat = bat_ref[...].reshape(1, BLK)
    gid = lax.broadcasted_iota(jnp.int32, (NUM_GRAPHS, BLK), 0)
    onehot_t = (gid == bat).astype(jnp.float32)
    gacc[...] += jnp.dot(onehot_t, h, preferred_element_type=jnp.float32)
    cacc[...] += jnp.sum(onehot_t, axis=1, keepdims=True)

    @pl.when(i == GRID - 1)
    def _():
        g = gacc[...] / jnp.maximum(cacc[...], 1.0)
        o_ref[...] = (
            jnp.dot(g, wl_ref[...], preferred_element_type=jnp.float32)
            + bl_ref[...]
        )


def _tc_pool(agg_a, agg_b, deg_a, deg_b, r, batf, w_lin, b_lin):
    return pl.pallas_call(
        _pool_body,
        grid=(GRID,),
        in_specs=[
            pl.BlockSpec((BLK, HID), lambda i: (i, 0)),
            pl.BlockSpec((BLK, HID), lambda i: (i, 0)),
            pl.BlockSpec((BLK, 16), lambda i: (i, 0)),
            pl.BlockSpec((BLK, 16), lambda i: (i, 0)),
            pl.BlockSpec((BLK, HID), lambda i: (i, 0)),
            pl.BlockSpec((1, 1, BLK), lambda i: (i, 0, 0)),
            pl.BlockSpec((HID, 1), lambda i: (0, 0)),
            pl.BlockSpec((1, 1), lambda i: (0, 0)),
        ],
        out_specs=pl.BlockSpec((NUM_GRAPHS, 1), lambda i: (0, 0)),
        out_shape=jax.ShapeDtypeStruct((NUM_GRAPHS, 1), jnp.float32),
        scratch_shapes=[
            pltpu.VMEM((NUM_GRAPHS, HID), jnp.float32),
            pltpu.VMEM((NUM_GRAPHS, 1), jnp.float32),
        ],
    )(agg_a, agg_b, deg_a, deg_b, r, batf, w_lin, b_lin)


def _sc_agg_call(p, src, dst, zeros64):
    mesh = plsc.VectorSubcoreMesh(core_axis_name="c", subcore_axis_name="s",
                                  num_cores=NC, num_subcores=NS)

    def body(p_h, src_h, dst_h, z64_h, out_h,
             srcv0, dstv0, msgv0, sem0, srcv1, dstv1, msgv1, sem1, acc):
        c = lax.axis_index("c")
        s = lax.axis_index("s")
        wid = c * NS + s
        base = wid * EW
        r0 = s * ROWS_PER_TILE

        pltpu.sync_copy(z64_h.at[pl.ds(r0, ROWS_PER_TILE)],
                        acc.at[pl.ds(r0, ROWS_PER_TILE)])
        plsc.subcore_barrier()

        def fetch(i, srcv, dstv, msgv, sem):
            off = base + i * CHUNK
            pltpu.sync_copy(src_h.at[pl.ds(off, CHUNK)], srcv)
            pltpu.sync_copy(dst_h.at[pl.ds(off, CHUNK)], dstv)
            return pltpu.async_copy(p_h.at[srcv], msgv, sem)

        fetch(0, srcv0, dstv0, msgv0, sem0)

        def two(j, carry):
            fetch(2 * j + 1, srcv1, dstv1, msgv1, sem1)
            pltpu.make_async_copy(p_h.at[srcv0], msgv0, sem0).wait()
            pltpu.sync_copy(msgv0, acc.at[dstv0], add=True)
            fetch(2 * j + 2, srcv0, dstv0, msgv0, sem0)
            pltpu.make_async_copy(p_h.at[srcv1], msgv1, sem1).wait()
            pltpu.sync_copy(msgv1, acc.at[dstv1], add=True)
            return carry

        lax.fori_loop(0, (NCH - 1) // 2, two, 0)
        pltpu.make_async_copy(p_h.at[srcv0], msgv0, sem0).wait()
        pltpu.sync_copy(msgv0, acc.at[dstv0], add=True)
        plsc.subcore_barrier()

        orow = c * NP + r0
        pltpu.sync_copy(acc.at[pl.ds(r0, ROWS_PER_TILE)],
                        out_h.at[pl.ds(orow, ROWS_PER_TILE)])

    fn = pl.kernel(
        body,
        out_type=[jax.ShapeDtypeStruct((NC * NP, HID), jnp.float32)],
        mesh=mesh,
        scratch_types=[
            pltpu.VMEM((CHUNK,), jnp.int32),
            pltpu.VMEM((CHUNK,), jnp.int32),
            pltpu.VMEM((CHUNK, HID), jnp.float32),
            pltpu.SemaphoreType.DMA,
            pltpu.VMEM((CHUNK,), jnp.int32),
            pltpu.VMEM((CHUNK,), jnp.int32),
            pltpu.VMEM((CHUNK, HID), jnp.float32),
            pltpu.SemaphoreType.DMA,
            pltpu.VMEM_SHARED((NP, HID), jnp.float32),
        ],
        compiler_params=pltpu.CompilerParams(use_tc_tiling_on_sc=False),
    )
    (out,) = fn(p, src, dst, zeros64)
    return out


DCHUNK = 2000


def _sc_degree_call(dst, zeros16, ones):
    mesh = plsc.VectorSubcoreMesh(core_axis_name="c", subcore_axis_name="s",
                                  num_cores=NC, num_subcores=NS)

    def body(dst_h, z16_h, ones_h, deg_h, dstv, onesv, dacc):
        c = lax.axis_index("c")
        s = lax.axis_index("s")
        wid = c * NS + s
        base = wid * EW
        r0 = s * ROWS_PER_TILE

        pltpu.sync_copy(z16_h.at[pl.ds(r0, ROWS_PER_TILE)],
                        dacc.at[pl.ds(r0, ROWS_PER_TILE)])
        pltpu.sync_copy(ones_h, onesv)
        plsc.subcore_barrier()

        def chunk(i, carry):
            off = base + i * DCHUNK
            pltpu.sync_copy(dst_h.at[pl.ds(off, DCHUNK)], dstv)
            pltpu.sync_copy(onesv, dacc.at[dstv], add=True)
            return carry

        lax.fori_loop(0, EW // DCHUNK, chunk, 0)
        plsc.subcore_barrier()

        orow = c * NP + r0
        pltpu.sync_copy(dacc.at[pl.ds(r0, ROWS_PER_TILE)],
                        deg_h.at[pl.ds(orow, ROWS_PER_TILE)])

    fn = pl.kernel(
        body,
        out_type=[jax.ShapeDtypeStruct((NC * NP, 16), jnp.float32)],
        mesh=mesh,
        scratch_types=[
            pltpu.VMEM((DCHUNK,), jnp.int32),
            pltpu.VMEM((DCHUNK, 16), jnp.float32),
            pltpu.VMEM_SHARED((NP, 16), jnp.float32),
        ],
        compiler_params=pltpu.CompilerParams(use_tc_tiling_on_sc=False),
    )
    (deg,) = fn(dst, zeros16, ones)
    return deg


def kernel(x, pos, edge_index, batch, W1_l, W1_r, b1, W2_l, W2_r, b2,
           W_lin, b_lin):
    f32 = jnp.float32
    src = edge_index[0].astype(jnp.int32)
    dst = edge_index[1].astype(jnp.int32)

    h0 = jnp.concatenate([x, pos], axis=1)
    d_in = h0.shape[1]
    kpad = 136
    h0p = jnp.pad(h0, ((0, 0), (0, kpad - d_in)))
    w1 = jnp.pad(jnp.concatenate([W1_l, W1_r], axis=1),
                 ((0, kpad - d_in), (0, 0)))
    b1cat = jnp.concatenate([jnp.zeros((HID,), f32), b1]).reshape(1, 2 * HID)
    out1 = _tc_matmul(h0p, w1, b1cat)
    p1 = out1[:, :HID]
    r1 = out1[:, HID:]

    zeros64 = jnp.zeros((NP, HID), f32)
    zeros16 = jnp.zeros((NP, 16), f32)
    ones = jnp.ones((DCHUNK, 16), f32)

    deg = _sc_degree_call(dst, zeros16, ones)
    deg_a, deg_b = deg[:N], deg[NP:NP + N]

    agg1 = _sc_agg_call(p1, src, dst, zeros64)

    w2 = jnp.concatenate([W2_l, W2_r], axis=1)
    b2cat = jnp.concatenate([jnp.zeros((HID,), f32), b2]).reshape(1, 2 * HID)
    out2 = _tc_combine_matmul(agg1[:N], agg1[NP:NP + N], deg_a, deg_b, r1,
                              w2, b2cat)
    p2 = out2[:, :HID]
    r2 = out2[:, HID:]

    agg2 = _sc_agg_call(p2, src, dst, zeros64)

    batf = batch.astype(jnp.int32).reshape(GRID, 1, BLK)
    out = _tc_pool(agg2[:N], agg2[NP:NP + N], deg_a, deg_b, r2, batf,
                   W_lin, b_lin.reshape(1, 1))
    return out


# speedup vs baseline: 17.1602x; 1.1790x over previous
import jax
import jax.numpy as jnp
from jax import lax
from jax.experimental import pallas as pl
from jax.experimental.pallas import tpu as pltpu
from jax.experimental.pallas import tpu_sc as plsc

N = 10000
E = 320000
HID = 64
NUM_GRAPHS = 128
NC = 2
NS = 16
EW = E // (NC * NS)
CHUNK = 400
NCH = EW // CHUNK
NP = 10240
ROWS_PER_TILE = NP // NS
BLK = 1000
GRID = N // BLK


def _mm_body(h_ref, w_ref, b_ref, p_ref, r_ref):
    res = (
        jnp.dot(h_ref[...], w_ref[...], preferred_element_type=jnp.float32)
        + b_ref[...]
    )
    p_ref[...] = res[:, :HID]
    r_ref[...] = res[:, HID:]


def _tc_matmul(h, w, b):
    m, k = h.shape
    return pl.pallas_call(
        _mm_body,
        grid=(m // BLK,),
        in_specs=[
            pl.BlockSpec((BLK, k), lambda i: (i, 0)),
            pl.BlockSpec((k, 2 * HID), lambda i: (0, 0)),
            pl.BlockSpec((1, 2 * HID), lambda i: (0, 0)),
        ],
        out_specs=[
            pl.BlockSpec((BLK, HID), lambda i: (i, 0)),
            pl.BlockSpec((BLK, HID), lambda i: (i, 0)),
        ],
        out_shape=[
            jax.ShapeDtypeStruct((m, HID), jnp.float32),
            jax.ShapeDtypeStruct((m, HID), jnp.float32),
        ],
    )(h, w, b)


def _mean_relu(agg, deg, r):
    d = deg[:, 0:1] + deg[:, 16:17]
    mean = (agg[:, :HID] + agg[:, HID:]) / jnp.maximum(d, 1.0)
    return jax.nn.relu(mean + r)


def _combine_mm_body(agg_ref, deg_ref, r_ref, w_ref, b_ref, p_ref, r2_ref):
    h = _mean_relu(agg_ref[...], deg_ref[...], r_ref[...])
    res = (
        jnp.dot(h, w_ref[...], preferred_element_type=jnp.float32) + b_ref[...]
    )
    p_ref[...] = res[:, :HID]
    r2_ref[...] = res[:, HID:]


def _tc_combine_matmul(agg, deg, r, w, b):
    return pl.pallas_call(
        _combine_mm_body,
        grid=(GRID,),
        in_specs=[
            pl.BlockSpec((BLK, 2 * HID), lambda i: (i, 0)),
            pl.BlockSpec((BLK, 32), lambda i: (i, 0)),
            pl.BlockSpec((BLK, HID), lambda i: (i, 0)),
            pl.BlockSpec((HID, 2 * HID), lambda i: (0, 0)),
            pl.BlockSpec((1, 2 * HID), lambda i: (0, 0)),
        ],
        out_specs=[
            pl.BlockSpec((BLK, HID), lambda i: (i, 0)),
            pl.BlockSpec((BLK, HID), lambda i: (i, 0)),
        ],
        out_shape=[
            jax.ShapeDtypeStruct((N, HID), jnp.float32),
            jax.ShapeDtypeStruct((N, HID), jnp.float32),
        ],
    )(agg, deg, r, w, b)


def _pool_body(agg_ref, deg_ref, r_ref, bat_ref, wl_ref, bl_ref,
               o_ref, gacc, cacc):
    i = pl.program_id(0)

    @pl.when(i == 0)
    def _():
        gacc[...] = jnp.zeros_like(gacc)
        cacc[...] = jnp.zeros_like(cacc)

    h = _mean_relu(agg_ref[...], deg_ref[...], r_ref[...])
    bat = bat_ref[...].reshape(1, BLK)
    gid = lax.broadcasted_iota(jnp.int32, (NUM_GRAPHS, BLK), 0)
    onehot_t = (gid == bat).astype(jnp.float32)
    gacc[...] += jnp.dot(onehot_t, h, preferred_element_type=jnp.float32)
    cacc[...] += jnp.sum(onehot_t, axis=1, keepdims=True)

    @pl.when(i == GRID - 1)
    def _():
        g = gacc[...] / jnp.maximum(cacc[...], 1.0)
        o_ref[...] = (
            jnp.dot(g, wl_ref[...], preferred_element_type=jnp.float32)
            + bl_ref[...]
        )


def _tc_pool(agg, deg, r, batf, w_lin, b_lin):
    return pl.pallas_call(
        _pool_body,
        grid=(GRID,),
        in_specs=[
            pl.BlockSpec((BLK, 2 * HID), lambda i: (i, 0)),
            pl.BlockSpec((BLK, 32), lambda i: (i, 0)),
            pl.BlockSpec((BLK, HID), lambda i: (i, 0)),
            pl.BlockSpec((1, 1, BLK), lambda i: (i, 0, 0)),
            pl.BlockSpec((HID, 1), lambda i: (0, 0)),
            pl.BlockSpec((1, 1), lambda i: (0, 0)),
        ],
        out_specs=pl.BlockSpec((NUM_GRAPHS, 1), lambda i: (0, 0)),
        out_shape=jax.ShapeDtypeStruct((NUM_GRAPHS, 1), jnp.float32),
        scratch_shapes=[
            pltpu.VMEM((NUM_GRAPHS, HID), jnp.float32),
            pltpu.VMEM((NUM_GRAPHS, 1), jnp.float32),
        ],
    )(agg, deg, r, batf, w_lin, b_lin)


def _sc_agg_call(p, src, dst, zeros64, zeros16=None, ones=None, with_deg=False):
    mesh = plsc.VectorSubcoreMesh(core_axis_name="c", subcore_axis_name="s",
                                  num_cores=NC, num_subcores=NS)

    out_type = [jax.ShapeDtypeStruct((NP, 2 * HID), jnp.float32)]
    scratch = [
        pltpu.VMEM((CHUNK,), jnp.int32),
        pltpu.VMEM((CHUNK,), jnp.int32),
        pltpu.VMEM((CHUNK, HID), jnp.float32),
        pltpu.SemaphoreType.DMA,
        pltpu.VMEM((CHUNK,), jnp.int32),
        pltpu.VMEM((CHUNK,), jnp.int32),
        pltpu.VMEM((CHUNK, HID), jnp.float32),
        pltpu.SemaphoreType.DMA,
        pltpu.VMEM_SHARED((NP, HID), jnp.float32),
    ]
    if with_deg:
        out_type.append(jax.ShapeDtypeStruct((NP, 32), jnp.float32))
        scratch += [
            pltpu.VMEM((CHUNK, 16), jnp.float32),
            pltpu.VMEM_SHARED((NP, 16), jnp.float32),
        ]

    def body(*refs):
        if with_deg:
            (p_h, src_h, dst_h, z64_h, z16_h, ones_h, out_h, deg_h,
             srcv0, dstv0, msgv0, sem0, srcv1, dstv1, msgv1, sem1, acc,
             onesv, dacc) = refs
        else:
            (p_h, src_h, dst_h, z64_h, out_h,
             srcv0, dstv0, msgv0, sem0, srcv1, dstv1, msgv1, sem1, acc) = refs
        c = lax.axis_index("c")
        s = lax.axis_index("s")
        wid = c * NS + s
        base = wid * EW
        r0 = s * ROWS_PER_TILE

        pltpu.sync_copy(z64_h.at[pl.ds(r0, ROWS_PER_TILE)],
                        acc.at[pl.ds(r0, ROWS_PER_TILE)])
        if with_deg:
            pltpu.sync_copy(z16_h.at[pl.ds(r0, ROWS_PER_TILE)],
                            dacc.at[pl.ds(r0, ROWS_PER_TILE)])
            pltpu.sync_copy(ones_h, onesv)
        plsc.subcore_barrier()

        def fetch(i, srcv, dstv, msgv, sem):
            off = base + i * CHUNK
            pltpu.sync_copy(src_h.at[pl.ds(off, CHUNK)], srcv)
            pltpu.sync_copy(dst_h.at[pl.ds(off, CHUNK)], dstv)
            pltpu.async_copy(p_h.at[srcv], msgv, sem)

        def drain_scatter(srcv, dstv, msgv, sem):
            pltpu.make_async_copy(p_h.at[srcv], msgv, sem).wait()
            pltpu.sync_copy(msgv, acc.at[dstv], add=True)
            if with_deg:
                pltpu.sync_copy(onesv, dacc.at[dstv], add=True)

        fetch(0, srcv0, dstv0, msgv0, sem0)

        def two(j, carry):
            fetch(2 * j + 1, srcv1, dstv1, msgv1, sem1)
            drain_scatter(srcv0, dstv0, msgv0, sem0)
            fetch(2 * j + 2, srcv0, dstv0, msgv0, sem0)
            drain_scatter(srcv1, dstv1, msgv1, sem1)
            return carry

        lax.fori_loop(0, (NCH - 1) // 2, two, 0)
        drain_scatter(srcv0, dstv0, msgv0, sem0)
        plsc.subcore_barrier()

        pltpu.sync_copy(acc.at[pl.ds(r0, ROWS_PER_TILE)],
                        out_h.at[pl.ds(r0, ROWS_PER_TILE),
                                 pl.ds(c * HID, HID)])
        if with_deg:
            pltpu.sync_copy(dacc.at[pl.ds(r0, ROWS_PER_TILE)],
                            deg_h.at[pl.ds(r0, ROWS_PER_TILE),
                                     pl.ds(c * 16, 16)])

    fn = pl.kernel(
        body, out_type=out_type, mesh=mesh, scratch_types=scratch,
        compiler_params=pltpu.CompilerParams(use_tc_tiling_on_sc=False),
    )
    if with_deg:
        return fn(p, src, dst, zeros64, zeros16, ones)
    return fn(p, src, dst, zeros64)


def kernel(x, pos, edge_index, batch, W1_l, W1_r, b1, W2_l, W2_r, b2,
           W_lin, b_lin):
    f32 = jnp.float32
    src = edge_index[0].astype(jnp.int32)
    dst = edge_index[1].astype(jnp.int32)

    h0 = jnp.concatenate([x, pos], axis=1)
    d_in = h0.shape[1]
    kpad = 136
    h0p = jnp.pad(h0, ((0, 0), (0, kpad - d_in)))
    w1 = jnp.pad(jnp.concatenate([W1_l, W1_r], axis=1),
                 ((0, kpad - d_in), (0, 0)))
    b1cat = jnp.concatenate([jnp.zeros((HID,), f32), b1]).reshape(1, 2 * HID)
    p1, r1 = _tc_matmul(h0p, w1, b1cat)

    zeros64 = jnp.zeros((NP, HID), f32)
    zeros16 = jnp.zeros((NP, 16), f32)
    ones = jnp.ones((CHUNK, 16), f32)

    agg1, deg = _sc_agg_call(p1, src, dst, zeros64, zeros16, ones, True)

    w2 = jnp.concatenate([W2_l, W2_r], axis=1)
    b2cat = jnp.concatenate([jnp.zeros((HID,), f32), b2]).reshape(1, 2 * HID)
    p2, r2 = _tc_combine_matmul(agg1, deg, r1, w2, b2cat)

    (agg2,) = _sc_agg_call(p2, src, dst, zeros64)

    batf = batch.astype(jnp.int32).reshape(GRID, 1, BLK)
    out = _tc_pool(agg2, deg, r2, batf, W_lin, b_lin.reshape(1, 1))
    return out


# speedup vs baseline: 18.9532x; 1.1045x over previous
import jax
import jax.numpy as jnp
from jax import lax
from jax.experimental import pallas as pl
from jax.experimental.pallas import tpu as pltpu
from jax.experimental.pallas import tpu_sc as plsc

N = 10000
E = 320000
HID = 64
NUM_GRAPHS = 128
NC = 2
NS = 16
EW = E // (NC * NS)
CHUNK = 400
NCH = EW // CHUNK
NP = 10240
ROWS_PER_TILE = NP // NS
BLK = 1000
GRID = N // BLK


def _mm_body(h_ref, w_ref, b_ref, p_ref, r_ref):
    res = (
        jnp.dot(h_ref[...], w_ref[...], preferred_element_type=jnp.float32)
        + b_ref[...]
    )
    p_ref[...] = res[:, :HID]
    r_ref[...] = res[:, HID:]


def _tc_matmul(h, w, b):
    m, k = h.shape
    return pl.pallas_call(
        _mm_body,
        grid=(m // BLK,),
        in_specs=[
            pl.BlockSpec((BLK, k), lambda i: (i, 0)),
            pl.BlockSpec((k, 2 * HID), lambda i: (0, 0)),
            pl.BlockSpec((1, 2 * HID), lambda i: (0, 0)),
        ],
        out_specs=[
            pl.BlockSpec((BLK, HID), lambda i: (i, 0)),
            pl.BlockSpec((BLK, HID), lambda i: (i, 0)),
        ],
        out_shape=[
            jax.ShapeDtypeStruct((m, HID), jnp.float32),
            jax.ShapeDtypeStruct((m, HID), jnp.float32),
        ],
    )(h, w, b)


def _mean_relu(agg, deg, r):
    d = deg[:, 0:1] + deg[:, 16:17]
    mean = (agg[:, :HID] + agg[:, HID:]) / jnp.maximum(d, 1.0)
    return jax.nn.relu(mean + r)


def _combine_mm_body(agg_ref, deg_ref, r_ref, w_ref, b_ref, p_ref, r2_ref):
    h = _mean_relu(agg_ref[...], deg_ref[...], r_ref[...])
    res = (
        jnp.dot(h, w_ref[...], preferred_element_type=jnp.float32) + b_ref[...]
    )
    p_ref[...] = res[:, :HID]
    r2_ref[...] = res[:, HID:]


def _tc_combine_matmul(agg, deg, r, w, b):
    return pl.pallas_call(
        _combine_mm_body,
        grid=(GRID,),
        in_specs=[
            pl.BlockSpec((BLK, 2 * HID), lambda i: (i, 0)),
            pl.BlockSpec((BLK, 32), lambda i: (i, 0)),
            pl.BlockSpec((BLK, HID), lambda i: (i, 0)),
            pl.BlockSpec((HID, 2 * HID), lambda i: (0, 0)),
            pl.BlockSpec((1, 2 * HID), lambda i: (0, 0)),
        ],
        out_specs=[
            pl.BlockSpec((BLK, HID), lambda i: (i, 0)),
            pl.BlockSpec((BLK, HID), lambda i: (i, 0)),
        ],
        out_shape=[
            jax.ShapeDtypeStruct((N, HID), jnp.float32),
            jax.ShapeDtypeStruct((N, HID), jnp.float32),
        ],
    )(agg, deg, r, w, b)


def _pool_body(agg_ref, deg_ref, r_ref, bat_ref, wl_ref, bl_ref,
               o_ref, gacc, cacc):
    i = pl.program_id(0)

    @pl.when(i == 0)
    def _():
        gacc[...] = jnp.zeros_like(gacc)
        cacc[...] = jnp.zeros_like(cacc)

    h = _mean_relu(agg_ref[...], deg_ref[...], r_ref[...])
    bat = bat_ref[...].reshape(1, BLK)
    gid = lax.broadcasted_iota(jnp.int32, (NUM_GRAPHS, BLK), 0)
    onehot_t = (gid == bat).astype(jnp.float32)
    gacc[...] += jnp.dot(onehot_t, h, preferred_element_type=jnp.float32)
    cacc[...] += jnp.sum(onehot_t, axis=1, keepdims=True)

    @pl.when(i == GRID - 1)
    def _():
        g = gacc[...] / jnp.maximum(cacc[...], 1.0)
        o_ref[...] = (
            jnp.dot(g, wl_ref[...], preferred_element_type=jnp.float32)
            + bl_ref[...]
        )


def _tc_pool(agg, deg, r, batf, w_lin, b_lin):
    return pl.pallas_call(
        _pool_body,
        grid=(GRID,),
        in_specs=[
            pl.BlockSpec((BLK, 2 * HID), lambda i: (i, 0)),
            pl.BlockSpec((BLK, 32), lambda i: (i, 0)),
            pl.BlockSpec((BLK, HID), lambda i: (i, 0)),
            pl.BlockSpec((1, 1, BLK), lambda i: (i, 0, 0)),
            pl.BlockSpec((HID, 1), lambda i: (0, 0)),
            pl.BlockSpec((1, 1), lambda i: (0, 0)),
        ],
        out_specs=pl.BlockSpec((NUM_GRAPHS, 1), lambda i: (0, 0)),
        out_shape=jax.ShapeDtypeStruct((NUM_GRAPHS, 1), jnp.float32),
        scratch_shapes=[
            pltpu.VMEM((NUM_GRAPHS, HID), jnp.float32),
            pltpu.VMEM((NUM_GRAPHS, 1), jnp.float32),
        ],
    )(agg, deg, r, batf, w_lin, b_lin)


def _sc_agg_call(p, src, dst, zeros64, zeros16=None, ones=None, with_deg=False):
    mesh = plsc.VectorSubcoreMesh(core_axis_name="c", subcore_axis_name="s",
                                  num_cores=NC, num_subcores=NS)

    out_type = [jax.ShapeDtypeStruct((NP, 2 * HID), jnp.float32)]
    scratch = [
        pltpu.VMEM((NCH, CHUNK), jnp.int32),
        pltpu.VMEM((NCH, CHUNK), jnp.int32),
        pltpu.VMEM((CHUNK, HID), jnp.float32),
        pltpu.SemaphoreType.DMA,
        pltpu.VMEM((CHUNK, HID), jnp.float32),
        pltpu.SemaphoreType.DMA,
        pltpu.VMEM_SHARED((NP, HID), jnp.float32),
    ]
    if with_deg:
        out_type.append(jax.ShapeDtypeStruct((NP, 32), jnp.float32))
        scratch += [
            pltpu.VMEM((CHUNK, 16), jnp.float32),
            pltpu.VMEM_SHARED((NP, 16), jnp.float32),
        ]

    def body(*refs):
        if with_deg:
            (p_h, src_h, dst_h, z64_h, z16_h, ones_h, out_h, deg_h,
             srcv, dstv, msgv0, sem0, msgv1, sem1, acc, onesv, dacc) = refs
        else:
            (p_h, src_h, dst_h, z64_h, out_h,
             srcv, dstv, msgv0, sem0, msgv1, sem1, acc) = refs
        c = lax.axis_index("c")
        s = lax.axis_index("s")
        wid = c * NS + s
        r0 = s * ROWS_PER_TILE

        pltpu.sync_copy(src_h.at[wid], srcv)
        pltpu.sync_copy(dst_h.at[wid], dstv)
        pltpu.sync_copy(z64_h.at[pl.ds(r0, ROWS_PER_TILE)],
                        acc.at[pl.ds(r0, ROWS_PER_TILE)])
        if with_deg:
            pltpu.sync_copy(z16_h.at[pl.ds(r0, ROWS_PER_TILE)],
                            dacc.at[pl.ds(r0, ROWS_PER_TILE)])
            pltpu.sync_copy(ones_h, onesv)
        plsc.subcore_barrier()

        def gather(i, msgv, sem):
            pltpu.async_copy(p_h.at[srcv.at[i]], msgv, sem)

        def drain_scatter(i, msgv, sem):
            pltpu.make_async_copy(p_h.at[srcv.at[i]], msgv, sem).wait()
            pltpu.sync_copy(msgv, acc.at[dstv.at[i]], add=True)
            if with_deg:
                pltpu.sync_copy(onesv, dacc.at[dstv.at[i]], add=True)

        gather(0, msgv0, sem0)

        def two(j, carry):
            gather(2 * j + 1, msgv1, sem1)
            drain_scatter(2 * j, msgv0, sem0)
            gather(2 * j + 2, msgv0, sem0)
            drain_scatter(2 * j + 1, msgv1, sem1)
            return carry

        lax.fori_loop(0, (NCH - 1) // 2, two, 0)
        drain_scatter(NCH - 1, msgv0, sem0)
        plsc.subcore_barrier()

        pltpu.sync_copy(acc.at[pl.ds(r0, ROWS_PER_TILE)],
                        out_h.at[pl.ds(r0, ROWS_PER_TILE),
                                 pl.ds(c * HID, HID)])
        if with_deg:
            pltpu.sync_copy(dacc.at[pl.ds(r0, ROWS_PER_TILE)],
                            deg_h.at[pl.ds(r0, ROWS_PER_TILE),
                                     pl.ds(c * 16, 16)])

    fn = pl.kernel(
        body, out_type=out_type, mesh=mesh, scratch_types=scratch,
        compiler_params=pltpu.CompilerParams(use_tc_tiling_on_sc=False),
    )
    if with_deg:
        return fn(p, src, dst, zeros64, zeros16, ones)
    return fn(p, src, dst, zeros64)


def kernel(x, pos, edge_index, batch, W1_l, W1_r, b1, W2_l, W2_r, b2,
           W_lin, b_lin):
    f32 = jnp.float32
    src = edge_index[0].astype(jnp.int32).reshape(NC * NS, NCH, CHUNK)
    dst = edge_index[1].astype(jnp.int32).reshape(NC * NS, NCH, CHUNK)

    h0 = jnp.concatenate([x, pos], axis=1)
    d_in = h0.shape[1]
    kpad = 136
    h0p = jnp.pad(h0, ((0, 0), (0, kpad - d_in)))
    w1 = jnp.pad(jnp.concatenate([W1_l, W1_r], axis=1),
                 ((0, kpad - d_in), (0, 0)))
    b1cat = jnp.concatenate([jnp.zeros((HID,), f32), b1]).reshape(1, 2 * HID)
    p1, r1 = _tc_matmul(h0p, w1, b1cat)

    zeros64 = jnp.zeros((NP, HID), f32)
    zeros16 = jnp.zeros((NP, 16), f32)
    ones = jnp.ones((CHUNK, 16), f32)

    agg1, deg = _sc_agg_call(p1, src, dst, zeros64, zeros16, ones, True)

    w2 = jnp.concatenate([W2_l, W2_r], axis=1)
    b2cat = jnp.concatenate([jnp.zeros((HID,), f32), b2]).reshape(1, 2 * HID)
    p2, r2 = _tc_combine_matmul(agg1, deg, r1, w2, b2cat)

    (agg2,) = _sc_agg_call(p2, src, dst, zeros64)

    batf = batch.astype(jnp.int32).reshape(GRID, 1, BLK)
    out = _tc_pool(agg2, deg, r2, batf, W_lin, b_lin.reshape(1, 1))
    return out
